# baseline (speedup 1.0000x reference)
import jax
import jax.numpy as jnp
from jax import lax
from jax.experimental import pallas as pl
from jax.experimental.pallas import tpu as pltpu
from jax.experimental.pallas import tpu_sc as plsc

_ROWS = 8192
_EMB = 1024
_NUM_WORKERS = 32


def _copy_body(table_hbm, out_hbm):
    c = lax.axis_index("c")
    s = lax.axis_index("s")
    wid = s * 2 + c
    rows_per_worker = _ROWS // _NUM_WORKERS
    base = wid * rows_per_worker
    pltpu.sync_copy(
        table_hbm.at[pl.ds(base, rows_per_worker)],
        out_hbm.at[pl.ds(base, rows_per_worker)],
    )


def kernel(x, table):
    del x
    mesh = plsc.VectorSubcoreMesh(core_axis_name="c", subcore_axis_name="s")
    out = pl.kernel(
        _copy_body,
        out_type=jax.ShapeDtypeStruct((_ROWS, _EMB), jnp.float32),
        mesh=mesh,
    )(table)
    return out[None]


# speedup vs baseline: 24.2617x; 24.2617x over previous
import jax
import jax.numpy as jnp
from jax import lax
from jax.experimental import pallas as pl
from jax.experimental.pallas import tpu as pltpu
from jax.experimental.pallas import tpu_sc as plsc

_ROWS = 8192
_EMB = 1024
_NUM_WORKERS = 32
_ROWS_PER_WORKER = _ROWS // _NUM_WORKERS
_CHUNK = 32
_NCHUNK = _ROWS_PER_WORKER // _CHUNK


def _copy_body(table_hbm, out_hbm, buf0, buf1, si0, si1, so0, so1):
    c = lax.axis_index("c")
    s = lax.axis_index("s")
    wid = s * 2 + c
    base = wid * _ROWS_PER_WORKER
    bufs = (buf0, buf1)
    sis = (si0, si1)
    sos = (so0, so1)

    def chunk(ref, i):
        return ref.at[pl.ds(base + i * _CHUNK, _CHUNK)]

    ins = [
        pltpu.make_async_copy(chunk(table_hbm, i), bufs[i % 2], sis[i % 2])
        for i in range(_NCHUNK)
    ]
    outs = [
        pltpu.make_async_copy(bufs[i % 2], chunk(out_hbm, i), sos[i % 2])
        for i in range(_NCHUNK)
    ]

    ins[0].start()
    for i in range(_NCHUNK):
        if i + 1 < _NCHUNK:
            if i >= 1:
                outs[i - 1].wait()
            ins[i + 1].start()
        ins[i].wait()
        outs[i].start()
    outs[_NCHUNK - 2].wait()
    outs[_NCHUNK - 1].wait()


def kernel(x, table):
    del x
    mesh = plsc.VectorSubcoreMesh(core_axis_name="c", subcore_axis_name="s")
    out = pl.kernel(
        _copy_body,
        out_type=jax.ShapeDtypeStruct((_ROWS, _EMB), jnp.float32),
        mesh=mesh,
        scratch_types=[
            pltpu.VMEM((_CHUNK, _EMB), jnp.float32),
            pltpu.VMEM((_CHUNK, _EMB), jnp.float32),
            pltpu.SemaphoreType.DMA,
            pltpu.SemaphoreType.DMA,
            pltpu.SemaphoreType.DMA,
            pltpu.SemaphoreType.DMA,
        ],
    )(table)
    return out[None]


# speedup vs baseline: 24.7057x; 1.0183x over previous
import jax
import jax.numpy as jnp
from jax import lax
from jax.experimental import pallas as pl
from jax.experimental.pallas import tpu as pltpu
from jax.experimental.pallas import tpu_sc as plsc

_ROWS = 8192
_EMB = 1024
_NUM_WORKERS = 32
_ROWS_PER_WORKER = _ROWS // _NUM_WORKERS
_CHUNK = 32
_NCHUNK = _ROWS_PER_WORKER // _CHUNK
_NBUF = 3


def _copy_body(table_hbm, out_hbm, *scratch):
    bufs = scratch[:_NBUF]
    sis = scratch[_NBUF:2 * _NBUF]
    sos = scratch[2 * _NBUF:]
    c = lax.axis_index("c")
    s = lax.axis_index("s")
    wid = s * 2 + c
    base = wid * _ROWS_PER_WORKER

    def chunk(ref, i):
        return ref.at[pl.ds(base + i * _CHUNK, _CHUNK)]

    ins = [
        pltpu.make_async_copy(chunk(table_hbm, i), bufs[i % _NBUF], sis[i % _NBUF])
        for i in range(_NCHUNK)
    ]
    outs = [
        pltpu.make_async_copy(bufs[i % _NBUF], chunk(out_hbm, i), sos[i % _NBUF])
        for i in range(_NCHUNK)
    ]

    for i in range(-(_NBUF - 1), _NCHUNK):
        k = i + _NBUF - 1
        if 0 <= k < _NCHUNK:
            if k >= _NBUF:
                outs[k - _NBUF].wait()
            ins[k].start()
        if i >= 0:
            ins[i].wait()
            outs[i].start()
    for i in range(max(0, _NCHUNK - _NBUF), _NCHUNK):
        outs[i].wait()


def kernel(x, table):
    del x
    mesh = plsc.VectorSubcoreMesh(core_axis_name="c", subcore_axis_name="s")
    out = pl.kernel(
        _copy_body,
        out_type=jax.ShapeDtypeStruct((_ROWS, _EMB), jnp.float32),
        mesh=mesh,
        scratch_types=(
            [pltpu.VMEM((_CHUNK, _EMB), jnp.float32)] * _NBUF
            + [pltpu.SemaphoreType.DMA] * (2 * _NBUF)
        ),
    )(table)
    return out[None]
